# baseline (speedup 1.0000x reference)
import jax
import jax.numpy as jnp
from jax.experimental import pallas as pl
from jax.experimental.pallas import tpu as pltpu

EPS_ = 1e-05
K_ = 256
BS_ = 512


def _chunk_sum(q):
    b = q[:, 0:8]
    for k in range(1, 16):
        b = b + q[:, 8 * k:8 * k + 8]
    e0 = b[:, 0:1] + b[:, 4:5]
    e1 = b[:, 2:3] + b[:, 6:7]
    e2 = b[:, 1:2] + b[:, 5:6]
    e3 = b[:, 3:4] + b[:, 7:8]
    return (e0 + e1) + (e2 + e3)


def _norm2(x):
    s = x * x
    acc = None
    for j in range(3):
        q = s[:, 256 * j:256 * j + 128] + s[:, 256 * j + 128:256 * j + 256]
        c = _chunk_sum(q)
        acc = c if acc is None else acc + c
    return acc


def _rowdot(p):
    acc = None
    for c in range(6):
        sc = _chunk_sum(p[:, 128 * c:128 * c + 128])
        acc = sc if acc is None else acc + sc
    return acc


def _sims_body(a_ref, b_ref, o_ref):
    a = a_ref[0]
    nxt = jnp.concatenate([a[1:], b_ref[0, 0:1]], axis=0)
    an = jnp.sqrt(_norm2(a)) + EPS_
    av = a / an
    nn = jnp.sqrt(_norm2(nxt)) + EPS_
    nv = nxt / nn
    s = jnp.abs(_rowdot(av * nv))
    o_ref[0, 0, :] = s[:, 0]


def _similarities(video):
    B, T, D = video.shape
    nj = T // BS_
    out = pl.pallas_call(
        _sims_body,
        grid=(B, nj),
        in_specs=[
            pl.BlockSpec((1, BS_, D), lambda b, j: (b, j, 0)),
            pl.BlockSpec((1, 8, D),
                         lambda b, j: (b, jnp.minimum(j + 1, nj - 1) * (BS_ // 8), 0)),
        ],
        out_specs=pl.BlockSpec((1, 1, BS_), lambda b, j: (b * nj + j, 0, 0)),
        out_shape=jax.ShapeDtypeStruct((B * nj, 1, BS_), jnp.float32),
    )(video, video)
    return out.reshape(B, T)


def kernel(video, audio):
    B, T, D = video.shape
    sims = _similarities(video)
    sims = sims.at[:, T - 1].set(jnp.inf)
    _, indices = jax.lax.top_k(-sims[:, : T - 1], K_)
    indices = indices + 1
    zeros = jnp.zeros((B, 1), dtype=indices.dtype)
    indices = jnp.concatenate([zeros, indices], axis=1)
    result_video = jnp.take_along_axis(video, indices[:, :, None], axis=1)
    result_audio = jnp.take_along_axis(audio, indices[:, :, None], axis=1)
    return (result_video, result_audio)


# speedup vs baseline: 6.0466x; 6.0466x over previous
import jax
import jax.numpy as jnp
from jax.experimental import pallas as pl
from jax.experimental.pallas import tpu as pltpu

EPS_ = 1e-05
K_ = 256
BS_ = 512


def _chunk_sum(q):
    b = q[:, 0:8]
    for k in range(1, 16):
        b = b + q[:, 8 * k:8 * k + 8]
    e0 = b[:, 0:1] + b[:, 4:5]
    e1 = b[:, 2:3] + b[:, 6:7]
    e2 = b[:, 1:2] + b[:, 5:6]
    e3 = b[:, 3:4] + b[:, 7:8]
    return (e0 + e1) + (e2 + e3)


def _norm2(x):
    s = x * x
    acc = None
    for j in range(3):
        q = s[:, 256 * j:256 * j + 128] + s[:, 256 * j + 128:256 * j + 256]
        c = _chunk_sum(q)
        acc = c if acc is None else acc + c
    return acc


def _rowdot(p):
    acc = None
    for c in range(6):
        sc = _chunk_sum(p[:, 128 * c:128 * c + 128])
        acc = sc if acc is None else acc + sc
    return acc


def _tile_sum(xt):
    b = xt[0:8, :]
    for k in range(1, 16):
        b = b + xt[8 * k:8 * k + 8, :]
    e0 = b[0:1, :] + b[4:5, :]
    e1 = b[2:3, :] + b[6:7, :]
    e2 = b[1:2, :] + b[5:6, :]
    e3 = b[3:4, :] + b[7:8, :]
    return (e0 + e1) + (e2 + e3)


def _norm2_lanes(x):
    s = x * x
    q = [s[:, 256 * j:256 * j + 128] + s[:, 256 * j + 128:256 * j + 256]
         for j in range(3)]
    groups = []
    for g in range(x.shape[0] // 128):
        acc = None
        for j in range(3):
            c = _tile_sum(q[j][128 * g:128 * g + 128, :].T)
            acc = c if acc is None else acc + c
        groups.append(acc)
    return jnp.concatenate(groups, axis=0)


def _rowdot_lanes(p):
    groups = []
    for g in range(p.shape[0] // 128):
        acc = None
        for c in range(6):
            sc = _tile_sum(p[128 * g:128 * g + 128, 128 * c:128 * c + 128].T)
            acc = sc if acc is None else acc + sc
        groups.append(acc)
    return jnp.concatenate(groups, axis=0)


def _sims_body(a_ref, b_ref, o_ref):
    a = a_ref[0]
    b0 = b_ref[0, 0:1]
    n2l = _norm2_lanes(a)
    an = jnp.sqrt(n2l) + EPS_
    an_col = jnp.concatenate(
        [an[g:g + 1, :].T for g in range(BS_ // 128)], axis=0)
    av = a / an_col
    bn = jnp.sqrt(_norm2(b0)) + EPS_
    bv = b0 / bn
    nxt_v = jnp.concatenate([av[1:], bv], axis=0)
    dl = _rowdot_lanes(av * nxt_v)
    s = jnp.abs(dl)
    for g in range(BS_ // 128):
        o_ref[0, 0, 128 * g:128 * (g + 1)] = s[g, :]


def _similarities(video):
    B, T, D = video.shape
    nj = T // BS_
    out = pl.pallas_call(
        _sims_body,
        grid=(B, nj),
        in_specs=[
            pl.BlockSpec((1, BS_, D), lambda b, j: (b, j, 0)),
            pl.BlockSpec((1, 8, D),
                         lambda b, j: (b, jnp.minimum(j + 1, nj - 1) * (BS_ // 8), 0)),
        ],
        out_specs=pl.BlockSpec((1, 1, BS_), lambda b, j: (b * nj + j, 0, 0)),
        out_shape=jax.ShapeDtypeStruct((B * nj, 1, BS_), jnp.float32),
    )(video, video)
    return out.reshape(B, T)


def kernel(video, audio):
    B, T, D = video.shape
    sims = _similarities(video)
    sims = sims.at[:, T - 1].set(jnp.inf)
    _, indices = jax.lax.top_k(-sims[:, : T - 1], K_)
    indices = indices + 1
    zeros = jnp.zeros((B, 1), dtype=indices.dtype)
    indices = jnp.concatenate([zeros, indices], axis=1)
    result_video = jnp.take_along_axis(video, indices[:, :, None], axis=1)
    result_audio = jnp.take_along_axis(audio, indices[:, :, None], axis=1)
    return (result_video, result_audio)
